# Initial kernel scaffold; baseline (speedup 1.0000x reference)
import jax
import jax.numpy as jnp
from jax.experimental import pallas as pl


def kernel(x, embeddings, nodes, edges, params, proj, graph):
    raise NotImplementedError("write your pallas kernel here")


# speedup vs baseline: 4.7022x; 4.7022x over previous
import functools

import jax
import jax.numpy as jnp
from jax import lax
from jax.experimental import pallas as pl
from jax.experimental.pallas import tpu as pltpu
from jax.experimental.pallas import tpu_sc as plsc

LATENT = 64
EMB = 16
KC = 1024
BG_K = 4
EPS = 1e-8

NB = 400
EB = 2000
NW = 32
GC = 1000
SC_C = 400
NH = 25000
TPT = 1576
NPADT = 16 * TPT
DUMP = 25152

F32 = jnp.float32
BF16 = jnp.bfloat16


def _bdot(a, b, dims=None):
    if dims is None:
        dims = (((a.ndim - 1,), (0,)), ((), ()))
    return lax.dot_general(a.astype(BF16), b.astype(BF16), dims,
                           preferred_element_type=F32)


def _hdot(a, b, dims=None):
    if dims is None:
        dims = (((a.ndim - 1,), (0,)), ((), ()))
    return lax.dot_general(a, b, dims, precision=lax.Precision.HIGHEST,
                           preferred_element_type=F32)


def _ln(h):
    m = jnp.mean(h, axis=-1, keepdims=True)
    v = jnp.mean((h - m) ** 2, axis=-1, keepdims=True)
    return (h - m) / jnp.sqrt(v + 1e-5)


def _relu_ln(h):
    return jnp.maximum(_ln(h), 0.0)


def _ka_body(emb_ref, proj_ref, cl_ref, means_ref, msq_ref, seg_acc, cnt_acc):
    i = pl.program_id(0)

    @pl.when(i == 0)
    def _():
        seg_acc[...] = jnp.zeros_like(seg_acc)
        cnt_acc[...] = jnp.zeros_like(cnt_acc)

    emb = emb_ref[...]
    s = _bdot(emb, proj_ref[...])
    v = jnp.max(s, axis=1, keepdims=True)
    iota = lax.broadcasted_iota(jnp.int32, (NB, KC), 1)
    idx = jnp.min(jnp.where(s >= v, iota, jnp.int32(2 ** 30)), axis=1)
    cl_ref[0, 0, :] = idx
    hot = (iota == idx[:, None]).astype(F32)
    seg_acc[...] += _hdot(hot, emb, (((0,), (0,)), ((), ())))
    cnt_acc[...] += _hdot(hot, jnp.ones((NB, 1), F32), (((0,), (0,)), ((), ())))

    @pl.when(i == pl.num_programs(0) - 1)
    def _():
        cnt = jnp.maximum(cnt_acc[...], 1.0)
        m = seg_acc[...] / cnt
        nrm = jnp.sqrt(jnp.sum(m * m, axis=1, keepdims=True))
        mn = m / (nrm + 1e-12)
        means_ref[...] = mn
        msq_ref[...] = jnp.sum(mn * mn, axis=1, keepdims=True)


def _run_ka(embeddings, proj):
    n = embeddings.shape[0]
    grid = n // NB
    cl3, means, msq = pl.pallas_call(
        _ka_body,
        grid=(grid,),
        in_specs=[
            pl.BlockSpec((NB, EMB), lambda i: (i, 0)),
            pl.BlockSpec((EMB, KC), lambda i: (0, 0)),
        ],
        out_specs=[
            pl.BlockSpec((1, 1, NB), lambda i: (i, 0, 0)),
            pl.BlockSpec((KC, EMB), lambda i: (0, 0)),
            pl.BlockSpec((KC, 1), lambda i: (0, 0)),
        ],
        out_shape=[
            jax.ShapeDtypeStruct((grid, 1, NB), jnp.int32),
            jax.ShapeDtypeStruct((KC, EMB), F32),
            jax.ShapeDtypeStruct((KC, 1), F32),
        ],
        scratch_shapes=[
            pltpu.VMEM((KC, EMB), F32),
            pltpu.VMEM((KC, 1), F32),
        ],
    )(embeddings, proj)
    return cl3.reshape(n), means, msq


def _kb_body(emb_ref, nodes_ref, means_ref, msq_ref, wb_ref,
             bidx_ref, bw_ref, sun_ref, den_ref):
    i = pl.program_id(0)

    @pl.when(i == 0)
    def _():
        sun_ref[...] = jnp.zeros_like(sun_ref)
        den_ref[...] = jnp.zeros_like(den_ref)

    emb = emb_ref[...]
    means = means_ref[...]
    st = _hdot(emb, means, (((1,), (1,)), ((), ())))
    s = _bdot(emb, means, (((1,), (1,)), ((), ())))
    esq = jnp.sum(emb * emb, axis=1, keepdims=True)
    iota = lax.broadcasted_iota(jnp.int32, (NB, KC), 1)
    wb = wb_ref[0, 0]

    sm = s
    facc = jnp.zeros((NB, KC), F32)
    idx_cols = []
    bw_cols = []
    for _ in range(BG_K):
        v = jnp.max(sm, axis=1, keepdims=True)
        idxk = jnp.min(jnp.where(sm >= v, iota, jnp.int32(2 ** 30)),
                       axis=1, keepdims=True)
        hot = (iota == idxk).astype(F32)
        mq = _hdot(hot, msq_ref[...])
        vt = jnp.sum(hot * st, axis=1, keepdims=True)
        d2 = esq + mq - 2.0 * vt
        bwk = jnp.exp(-wb * d2)
        facc = facc + bwk * hot
        idx_cols.append(idxk)
        bw_cols.append(bwk)
        sm = jnp.where(iota == idxk, -jnp.inf, sm)

    bidx_ref[...] = jnp.concatenate(idx_cols, axis=1)
    bw_ref[...] = jnp.concatenate(bw_cols, axis=1)
    sun_ref[...] += _hdot(facc, nodes_ref[...], (((0,), (0,)), ((), ())))
    den_ref[...] += _hdot(facc, jnp.ones((NB, 1), F32), (((0,), (0,)), ((), ())))


def _run_kb(embeddings, nodes, means, msq, wb):
    n = embeddings.shape[0]
    grid = n // NB
    return pl.pallas_call(
        _kb_body,
        grid=(grid,),
        in_specs=[
            pl.BlockSpec((NB, EMB), lambda i: (i, 0)),
            pl.BlockSpec((NB, LATENT), lambda i: (i, 0)),
            pl.BlockSpec((KC, EMB), lambda i: (0, 0)),
            pl.BlockSpec((KC, 1), lambda i: (0, 0)),
            pl.BlockSpec((1, 1), lambda i: (0, 0)),
        ],
        out_specs=[
            pl.BlockSpec((NB, BG_K), lambda i: (i, 0)),
            pl.BlockSpec((NB, BG_K), lambda i: (i, 0)),
            pl.BlockSpec((KC, LATENT), lambda i: (0, 0)),
            pl.BlockSpec((KC, 1), lambda i: (0, 0)),
        ],
        out_shape=[
            jax.ShapeDtypeStruct((n, BG_K), jnp.int32),
            jax.ShapeDtypeStruct((n, BG_K), F32),
            jax.ShapeDtypeStruct((KC, LATENT), F32),
            jax.ShapeDtypeStruct((KC, 1), F32),
        ],
    )(embeddings, nodes, means, msq, wb)


def _kc_body(sun_ref, den_ref, means_ref, w1_ref, b1_ref, w2_ref, b2_ref,
             sup_ref):
    inv = 1.0 / (den_ref[...] + EPS)
    sn_raw = sun_ref[...] * inv
    h1 = _relu_ln(_bdot(sn_raw, w1_ref[...]) + b1_ref[...])
    h2 = _relu_ln(_bdot(h1, w2_ref[...]) + b2_ref[...])
    sup = jnp.concatenate([means_ref[...], h2], axis=1)
    sup_ref[...] = sup * inv


def _run_kc(sun, den, means, w1, b1, w2, b2):
    return pl.pallas_call(
        _kc_body,
        out_shape=jax.ShapeDtypeStruct((KC, LATENT), F32),
    )(sun, den, means, w1, b1, w2, b2)


def _kf_body(src_ref, dst_ref, edg_ref, w1_ref, b1_ref, w2_ref, b2_ref,
             out_ref):
    edg = edg_ref[...]
    h = jnp.concatenate([src_ref[...], dst_ref[...], edg], axis=1)
    h1 = _relu_ln(_bdot(h, w1_ref[...]) + b1_ref[...])
    h2 = _relu_ln(_bdot(h1, w2_ref[...]) + b2_ref[...])
    out_ref[...] = h2 + edg


def _run_kf(src, dst, edges, w1, b1, w2, b2):
    e = edges.shape[0]
    grid = e // EB
    return pl.pallas_call(
        _kf_body,
        grid=(grid,),
        in_specs=[
            pl.BlockSpec((EB, LATENT), lambda i: (i, 0)),
            pl.BlockSpec((EB, LATENT), lambda i: (i, 0)),
            pl.BlockSpec((EB, LATENT), lambda i: (i, 0)),
            pl.BlockSpec((3 * LATENT, LATENT), lambda i: (0, 0)),
            pl.BlockSpec((1, LATENT), lambda i: (0, 0)),
            pl.BlockSpec((LATENT, LATENT), lambda i: (0, 0)),
            pl.BlockSpec((1, LATENT), lambda i: (0, 0)),
        ],
        out_specs=pl.BlockSpec((EB, LATENT), lambda i: (i, 0)),
        out_shape=jax.ShapeDtypeStruct((e, LATENT), F32),
    )(src, dst, edges, w1, b1, w2, b2)


def _kd_body(nodes_ref, agg_ref, bidx_ref, bw_ref, sup_ref,
             w1_ref, b1_ref, w2_ref, b2_ref,
             ow1_ref, ob1_ref, ow2_ref, ob2_ref, out_ref):
    iota = lax.broadcasted_iota(jnp.int32, (NB, KC), 1)
    bidx = bidx_ref[...]
    bw = bw_ref[...]
    facc = jnp.zeros((NB, KC), F32)
    for k in range(BG_K):
        hot = (iota == bidx[:, k:k + 1]).astype(F32)
        facc = facc + bw[:, k:k + 1] * hot
    from_super = _hdot(facc, sup_ref[...])
    nodes = nodes_ref[...]
    h = jnp.concatenate([nodes, agg_ref[...], from_super], axis=1)
    h1 = _relu_ln(_bdot(h, w1_ref[...]) + b1_ref[...])
    h2 = _relu_ln(_bdot(h1, w2_ref[...]) + b2_ref[...])
    nn = h2 + nodes
    o1 = _relu_ln(_bdot(nn, ow1_ref[...]) + ob1_ref[...])
    o2 = _bdot(o1, ow2_ref[...]) + ob2_ref[...]
    nrm = jnp.sqrt(jnp.sum(o2 * o2, axis=1, keepdims=True))
    out_ref[...] = o2 / (nrm + 1e-12)


def _run_kd(nodes, agg, bidx, bw, sup, w1, b1, w2, b2, ow1, ob1, ow2, ob2):
    n = nodes.shape[0]
    grid = n // NB
    return pl.pallas_call(
        _kd_body,
        grid=(grid,),
        in_specs=[
            pl.BlockSpec((NB, LATENT), lambda i: (i, 0)),
            pl.BlockSpec((NB, LATENT), lambda i: (i, 0)),
            pl.BlockSpec((NB, BG_K), lambda i: (i, 0)),
            pl.BlockSpec((NB, BG_K), lambda i: (i, 0)),
            pl.BlockSpec((KC, LATENT), lambda i: (0, 0)),
            pl.BlockSpec((3 * LATENT, LATENT), lambda i: (0, 0)),
            pl.BlockSpec((1, LATENT), lambda i: (0, 0)),
            pl.BlockSpec((LATENT, LATENT), lambda i: (0, 0)),
            pl.BlockSpec((1, LATENT), lambda i: (0, 0)),
            pl.BlockSpec((LATENT, LATENT), lambda i: (0, 0)),
            pl.BlockSpec((1, LATENT), lambda i: (0, 0)),
            pl.BlockSpec((LATENT, EMB), lambda i: (0, 0)),
            pl.BlockSpec((1, EMB), lambda i: (0, 0)),
        ],
        out_specs=pl.BlockSpec((NB, EMB), lambda i: (i, 0)),
        out_shape=jax.ShapeDtypeStruct((n, EMB), F32),
    )(nodes, agg, bidx, bw, sup, w1, b1, w2, b2, ow1, ob1, ow2, ob2)


def _make_gather(e_total, d):
    per_w = e_total // NW
    iters = per_w // GC
    mesh = plsc.VectorSubcoreMesh(core_axis_name="c", subcore_axis_name="s")

    @functools.partial(
        pl.kernel, mesh=mesh,
        out_type=jax.ShapeDtypeStruct((e_total, d), F32),
        scratch_types=[
            pltpu.VMEM((GC,), jnp.int32),
            pltpu.VMEM((GC, d), F32),
            pltpu.SemaphoreType.DMA,
        ],
        compiler_params=pltpu.CompilerParams(use_tc_tiling_on_sc=False),
    )
    def gk(table_hbm, idx_hbm, out_hbm, idx_v, rows_v, sem):
        wid = lax.axis_index("s") * 2 + lax.axis_index("c")

        def body(j, carry):
            base = wid * per_w + j * GC
            pltpu.sync_copy(idx_hbm.at[pl.ds(base, GC)], idx_v)
            pltpu.async_copy(table_hbm.at[idx_v], rows_v, sem).wait()
            pltpu.sync_copy(rows_v, out_hbm.at[pl.ds(base, GC)])
            return carry

        lax.fori_loop(0, iters, body, 0)

    return gk


def _make_scatter(e_total, d):
    per_t = e_total // 16
    iters = per_t // SC_C
    mesh = plsc.VectorSubcoreMesh(core_axis_name="c", subcore_axis_name="s")

    @functools.partial(
        pl.kernel, mesh=mesh,
        out_type=jax.ShapeDtypeStruct((2, NPADT, d), F32),
        scratch_types=[
            pltpu.VMEM((SC_C,), jnp.int32),
            pltpu.VMEM((SC_C,), jnp.int32),
            pltpu.VMEM((SC_C, d), F32),
            pltpu.VMEM_SHARED((NPADT, d), F32),
        ],
        compiler_params=pltpu.CompilerParams(use_tc_tiling_on_sc=False),
    )
    def sk(rows_hbm, g1_hbm, zeros_hbm, out_hbm, gidx_v, lidx_v, rows_v, shared):
        c = lax.axis_index("c")
        t = lax.axis_index("s")
        pltpu.sync_copy(zeros_hbm, shared.at[pl.ds(t * TPT, TPT)])
        plsc.subcore_barrier()
        lo = c * NH

        def body(j, carry):
            eoff = t * per_t + j * SC_C
            pltpu.sync_copy(g1_hbm.at[pl.ds(eoff, SC_C)], gidx_v)

            def ixb(i, cc):
                v = gidx_v[pl.ds(i * 16, 16)]
                l = v - lo
                m = (l >= 0) & (l < NH)
                lidx_v[pl.ds(i * 16, 16)] = jnp.where(m, l, jnp.int32(DUMP))
                return cc

            lax.fori_loop(0, SC_C // 16, ixb, 0)
            pltpu.sync_copy(rows_hbm.at[pl.ds(eoff, SC_C)], rows_v)
            pltpu.sync_copy(rows_v, shared.at[lidx_v], add=True)
            return carry

        lax.fori_loop(0, iters, body, 0)
        plsc.subcore_barrier()
        pltpu.sync_copy(shared.at[pl.ds(t * TPT, TPT)],
                        out_hbm.at[c, pl.ds(t * TPT, TPT)])

    return sk


def kernel(x, embeddings, nodes, edges, params, proj, graph):
    n = embeddings.shape[0]
    e = edges.shape[0]
    g0 = graph[0].astype(jnp.int32)
    g1 = graph[1].astype(jnp.int32)

    clusters, means, msq = _run_ka(embeddings, proj)

    gather = _make_gather(e, LATENT)
    src = gather(nodes, g0)
    dst = gather(nodes, g1)

    pe = params["cell_edge"]
    enew = _run_kf(src, dst, edges,
                   pe[0][0], pe[0][1].reshape(1, LATENT),
                   pe[1][0], pe[1][1].reshape(1, LATENT))

    scatter = _make_scatter(e, LATENT)
    agg2 = scatter(enew, g1, jnp.zeros((TPT, LATENT), F32))
    agg = agg2[:, :NH, :].reshape(n, LATENT)

    wb = jnp.abs(params["w_bip"]).reshape(1, 1)
    bidx, bw, sun, den = _run_kb(embeddings, nodes, means, msq, wb)

    ps = params["supernode_enc"]
    sup = _run_kc(sun, den, means,
                  ps[0][0], ps[0][1].reshape(1, 64),
                  ps[1][0], ps[1][1].reshape(1, LATENT - EMB))

    pn = params["cell_node"]
    po = params["output"]
    out = _run_kd(nodes, agg, bidx, bw, sup,
                  pn[0][0], pn[0][1].reshape(1, LATENT),
                  pn[1][0], pn[1][1].reshape(1, LATENT),
                  po[0][0], po[0][1].reshape(1, 64),
                  po[1][0], po[1][1].reshape(1, EMB))

    return (out, clusters)


# speedup vs baseline: 6.1936x; 1.3172x over previous
import functools

import jax
import jax.numpy as jnp
from jax import lax
from jax.experimental import pallas as pl
from jax.experimental.pallas import tpu as pltpu
from jax.experimental.pallas import tpu_sc as plsc

LATENT = 64
EMB = 16
KC = 1024
BG_K = 4
EPS = 1e-8

NB = 400
EB = 8000
NW = 32
GC = 1000
SC_C = 400
NH = 25000
TPT = 1576
NPADT = 16 * TPT
DUMP = 25152

F32 = jnp.float32
BF16 = jnp.bfloat16


def _bdot(a, b, dims=None):
    if dims is None:
        dims = (((a.ndim - 1,), (0,)), ((), ()))
    return lax.dot_general(a.astype(BF16), b.astype(BF16), dims,
                           preferred_element_type=F32)


def _hdot(a, b, dims=None):
    if dims is None:
        dims = (((a.ndim - 1,), (0,)), ((), ()))
    return lax.dot_general(a, b, dims, precision=lax.Precision.HIGHEST,
                           preferred_element_type=F32)


def _ln(h):
    m = jnp.mean(h, axis=-1, keepdims=True)
    v = jnp.mean((h - m) ** 2, axis=-1, keepdims=True)
    return (h - m) / jnp.sqrt(v + 1e-5)


def _relu_ln(h):
    return jnp.maximum(_ln(h), 0.0)


def _ka_body(emb_ref, proj_ref, cl_ref, means_ref, msq_ref, seg_acc, cnt_acc):
    i = pl.program_id(0)

    @pl.when(i == 0)
    def _():
        seg_acc[...] = jnp.zeros_like(seg_acc)
        cnt_acc[...] = jnp.zeros_like(cnt_acc)

    emb = emb_ref[...]
    s = _bdot(emb, proj_ref[...])
    v = jnp.max(s, axis=1, keepdims=True)
    iota = lax.broadcasted_iota(jnp.int32, (NB, KC), 1)
    idx = jnp.min(jnp.where(s >= v, iota, jnp.int32(2 ** 30)), axis=1)
    cl_ref[0, 0, :] = idx
    hot = (iota == idx[:, None]).astype(F32)
    seg_acc[...] += _hdot(hot, emb, (((0,), (0,)), ((), ())))
    cnt_acc[...] += _hdot(hot, jnp.ones((NB, 1), F32), (((0,), (0,)), ((), ())))

    @pl.when(i == pl.num_programs(0) - 1)
    def _():
        cnt = jnp.maximum(cnt_acc[...], 1.0)
        m = seg_acc[...] / cnt
        nrm = jnp.sqrt(jnp.sum(m * m, axis=1, keepdims=True))
        mn = m / (nrm + 1e-12)
        means_ref[...] = mn
        msq_ref[...] = jnp.sum(mn * mn, axis=1, keepdims=True)


def _run_ka(embeddings, proj):
    n = embeddings.shape[0]
    grid = n // NB
    cl3, means, msq = pl.pallas_call(
        _ka_body,
        grid=(grid,),
        in_specs=[
            pl.BlockSpec((NB, EMB), lambda i: (i, 0)),
            pl.BlockSpec((EMB, KC), lambda i: (0, 0)),
        ],
        out_specs=[
            pl.BlockSpec((1, 1, NB), lambda i: (i, 0, 0)),
            pl.BlockSpec((KC, EMB), lambda i: (0, 0)),
            pl.BlockSpec((KC, 1), lambda i: (0, 0)),
        ],
        out_shape=[
            jax.ShapeDtypeStruct((grid, 1, NB), jnp.int32),
            jax.ShapeDtypeStruct((KC, EMB), F32),
            jax.ShapeDtypeStruct((KC, 1), F32),
        ],
        scratch_shapes=[
            pltpu.VMEM((KC, EMB), F32),
            pltpu.VMEM((KC, 1), F32),
        ],
    )(embeddings, proj)
    return cl3.reshape(n), means, msq


def _kb_body(emb_ref, nodes_ref, means_ref, msqr_ref, wb_ref,
             bidx_ref, bw_ref, sun_ref, den_ref):
    i = pl.program_id(0)

    @pl.when(i == 0)
    def _():
        sun_ref[...] = jnp.zeros_like(sun_ref)
        den_ref[...] = jnp.zeros_like(den_ref)

    emb = emb_ref[...]
    means = means_ref[...]
    st = _hdot(emb, means, (((1,), (1,)), ((), ())))
    s = _bdot(emb, means, (((1,), (1,)), ((), ())))
    esq = jnp.sum(emb * emb, axis=1, keepdims=True)
    iota = lax.broadcasted_iota(jnp.int32, (NB, KC), 1)
    wb = wb_ref[0, 0]
    tmat = msqr_ref[...] - 2.0 * st

    sm = s
    facc = jnp.zeros((NB, KC), F32)
    idx_cols = []
    bw_cols = []
    for _ in range(BG_K):
        v = jnp.max(sm, axis=1, keepdims=True)
        idxk = jnp.min(jnp.where(sm >= v, iota, jnp.int32(2 ** 30)),
                       axis=1, keepdims=True)
        hot = (iota == idxk).astype(F32)
        q = jnp.sum(hot * tmat, axis=1, keepdims=True)
        d2 = esq + q
        bwk = jnp.exp(-wb * d2)
        facc = facc + bwk * hot
        idx_cols.append(idxk)
        bw_cols.append(bwk)
        sm = jnp.where(iota == idxk, -jnp.inf, sm)

    bidx_ref[...] = jnp.concatenate(idx_cols, axis=1)
    bw_ref[...] = jnp.concatenate(bw_cols, axis=1)
    sun_ref[...] += _bdot(facc, nodes_ref[...], (((0,), (0,)), ((), ())))
    den_ref[...] += jnp.sum(facc, axis=0, keepdims=True)


def _run_kb(embeddings, nodes, means, msq, wb):
    n = embeddings.shape[0]
    grid = n // NB
    return pl.pallas_call(
        _kb_body,
        grid=(grid,),
        in_specs=[
            pl.BlockSpec((NB, EMB), lambda i: (i, 0)),
            pl.BlockSpec((NB, LATENT), lambda i: (i, 0)),
            pl.BlockSpec((KC, EMB), lambda i: (0, 0)),
            pl.BlockSpec((1, KC), lambda i: (0, 0)),
            pl.BlockSpec((1, 1), lambda i: (0, 0)),
        ],
        out_specs=[
            pl.BlockSpec((NB, BG_K), lambda i: (i, 0)),
            pl.BlockSpec((NB, BG_K), lambda i: (i, 0)),
            pl.BlockSpec((KC, LATENT), lambda i: (0, 0)),
            pl.BlockSpec((1, KC), lambda i: (0, 0)),
        ],
        out_shape=[
            jax.ShapeDtypeStruct((n, BG_K), jnp.int32),
            jax.ShapeDtypeStruct((n, BG_K), F32),
            jax.ShapeDtypeStruct((KC, LATENT), F32),
            jax.ShapeDtypeStruct((1, KC), F32),
        ],
        compiler_params=pltpu.CompilerParams(fuse_transposed_lhs_in_matmul=True),
    )(embeddings, nodes, means, msq, wb)


def _kc_body(sun_ref, den_ref, means_ref, w1_ref, b1_ref, w2_ref, b2_ref,
             sup_ref):
    inv = 1.0 / (den_ref[...] + EPS)
    sn_raw = sun_ref[...] * inv
    h1 = _relu_ln(_bdot(sn_raw, w1_ref[...]) + b1_ref[...])
    h2 = _relu_ln(_bdot(h1, w2_ref[...]) + b2_ref[...])
    sup = jnp.concatenate([means_ref[...], h2], axis=1)
    sup_ref[...] = sup * inv


def _run_kc(sun, den, means, w1, b1, w2, b2):
    return pl.pallas_call(
        _kc_body,
        out_shape=jax.ShapeDtypeStruct((KC, LATENT), F32),
    )(sun, den, means, w1, b1, w2, b2)


def _kf_body(src_ref, dst_ref, edg_ref, w1a_ref, w1b_ref, w1c_ref, b1_ref,
             w2_ref, b2_ref, out_ref):
    edg = edg_ref[...]
    p = (_bdot(src_ref[...], w1a_ref[...]) + _bdot(dst_ref[...], w1b_ref[...])
         + _bdot(edg, w1c_ref[...]) + b1_ref[...])
    h1 = _relu_ln(p)
    h2 = _relu_ln(_bdot(h1, w2_ref[...]) + b2_ref[...])
    out_ref[...] = h2 + edg


def _run_kf(src, dst, edges, w1, b1, w2, b2):
    e = edges.shape[0]
    grid = e // EB
    return pl.pallas_call(
        _kf_body,
        grid=(grid,),
        in_specs=[
            pl.BlockSpec((EB, LATENT), lambda i: (i, 0)),
            pl.BlockSpec((EB, LATENT), lambda i: (i, 0)),
            pl.BlockSpec((EB, LATENT), lambda i: (i, 0)),
            pl.BlockSpec((LATENT, LATENT), lambda i: (0, 0)),
            pl.BlockSpec((LATENT, LATENT), lambda i: (0, 0)),
            pl.BlockSpec((LATENT, LATENT), lambda i: (0, 0)),
            pl.BlockSpec((1, LATENT), lambda i: (0, 0)),
            pl.BlockSpec((LATENT, LATENT), lambda i: (0, 0)),
            pl.BlockSpec((1, LATENT), lambda i: (0, 0)),
        ],
        out_specs=pl.BlockSpec((EB, LATENT), lambda i: (i, 0)),
        out_shape=jax.ShapeDtypeStruct((e, LATENT), F32),
    )(src, dst, edges, w1[:LATENT], w1[LATENT:2 * LATENT], w1[2 * LATENT:],
      b1, w2, b2)


def _kd_body(nodes_ref, agg_ref, bidx_ref, bw_ref, sup_ref,
             w1_ref, b1_ref, w2_ref, b2_ref,
             ow1_ref, ob1_ref, ow2_ref, ob2_ref, out_ref):
    iota = lax.broadcasted_iota(jnp.int32, (NB, KC), 1)
    bidx = bidx_ref[...]
    bw = bw_ref[...]
    facc = jnp.zeros((NB, KC), F32)
    for k in range(BG_K):
        hot = (iota == bidx[:, k:k + 1]).astype(F32)
        facc = facc + bw[:, k:k + 1] * hot
    from_super = _bdot(facc, sup_ref[...])
    nodes = nodes_ref[...]
    h = jnp.concatenate([nodes, agg_ref[...], from_super], axis=1)
    h1 = _relu_ln(_bdot(h, w1_ref[...]) + b1_ref[...])
    h2 = _relu_ln(_bdot(h1, w2_ref[...]) + b2_ref[...])
    nn = h2 + nodes
    o1 = _relu_ln(_bdot(nn, ow1_ref[...]) + ob1_ref[...])
    o2 = _bdot(o1, ow2_ref[...]) + ob2_ref[...]
    nrm = jnp.sqrt(jnp.sum(o2 * o2, axis=1, keepdims=True))
    out_ref[...] = o2 / (nrm + 1e-12)


def _run_kd(nodes, agg, bidx, bw, sup, w1, b1, w2, b2, ow1, ob1, ow2, ob2):
    n = nodes.shape[0]
    grid = n // NB
    return pl.pallas_call(
        _kd_body,
        grid=(grid,),
        in_specs=[
            pl.BlockSpec((NB, LATENT), lambda i: (i, 0)),
            pl.BlockSpec((NB, LATENT), lambda i: (i, 0)),
            pl.BlockSpec((NB, BG_K), lambda i: (i, 0)),
            pl.BlockSpec((NB, BG_K), lambda i: (i, 0)),
            pl.BlockSpec((KC, LATENT), lambda i: (0, 0)),
            pl.BlockSpec((3 * LATENT, LATENT), lambda i: (0, 0)),
            pl.BlockSpec((1, LATENT), lambda i: (0, 0)),
            pl.BlockSpec((LATENT, LATENT), lambda i: (0, 0)),
            pl.BlockSpec((1, LATENT), lambda i: (0, 0)),
            pl.BlockSpec((LATENT, LATENT), lambda i: (0, 0)),
            pl.BlockSpec((1, LATENT), lambda i: (0, 0)),
            pl.BlockSpec((LATENT, EMB), lambda i: (0, 0)),
            pl.BlockSpec((1, EMB), lambda i: (0, 0)),
        ],
        out_specs=pl.BlockSpec((NB, EMB), lambda i: (i, 0)),
        out_shape=jax.ShapeDtypeStruct((n, EMB), F32),
    )(nodes, agg, bidx, bw, sup, w1, b1, w2, b2, ow1, ob1, ow2, ob2)


def _make_gather(e_total, d):
    per_w = e_total // NW
    iters = per_w // GC
    mesh = plsc.VectorSubcoreMesh(core_axis_name="c", subcore_axis_name="s")

    @functools.partial(
        pl.kernel, mesh=mesh,
        out_type=jax.ShapeDtypeStruct((e_total, d), F32),
        scratch_types=[
            pltpu.VMEM((GC,), jnp.int32),
            pltpu.VMEM((GC, d), F32),
            pltpu.SemaphoreType.DMA,
        ],
        compiler_params=pltpu.CompilerParams(use_tc_tiling_on_sc=False),
    )
    def gk(table_hbm, idx_hbm, out_hbm, idx_v, rows_v, sem):
        wid = lax.axis_index("s") * 2 + lax.axis_index("c")

        def body(j, carry):
            base = wid * per_w + j * GC
            pltpu.sync_copy(idx_hbm.at[pl.ds(base, GC)], idx_v)
            pltpu.async_copy(table_hbm.at[idx_v], rows_v, sem).wait()
            pltpu.sync_copy(rows_v, out_hbm.at[pl.ds(base, GC)])
            return carry

        lax.fori_loop(0, iters, body, 0)

    return gk


def _make_scatter(e_total, d):
    per_t = e_total // 16
    iters = per_t // SC_C
    mesh = plsc.VectorSubcoreMesh(core_axis_name="c", subcore_axis_name="s")

    @functools.partial(
        pl.kernel, mesh=mesh,
        out_type=jax.ShapeDtypeStruct((2, NPADT, d), F32),
        scratch_types=[
            pltpu.VMEM((SC_C,), jnp.int32),
            pltpu.VMEM((SC_C,), jnp.int32),
            pltpu.VMEM((SC_C, d), F32),
            pltpu.VMEM_SHARED((NPADT, d), F32),
        ],
        compiler_params=pltpu.CompilerParams(use_tc_tiling_on_sc=False),
    )
    def sk(rows_hbm, g1_hbm, zeros_hbm, out_hbm, gidx_v, lidx_v, rows_v, shared):
        c = lax.axis_index("c")
        t = lax.axis_index("s")
        pltpu.sync_copy(zeros_hbm, shared.at[pl.ds(t * TPT, TPT)])
        plsc.subcore_barrier()
        lo = c * NH

        def body(j, carry):
            eoff = t * per_t + j * SC_C
            pltpu.sync_copy(g1_hbm.at[pl.ds(eoff, SC_C)], gidx_v)

            def ixb(i, cc):
                v = gidx_v[pl.ds(i * 16, 16)]
                l = v - lo
                m = (l >= 0) & (l < NH)
                lidx_v[pl.ds(i * 16, 16)] = jnp.where(m, l, jnp.int32(DUMP))
                return cc

            lax.fori_loop(0, SC_C // 16, ixb, 0)
            pltpu.sync_copy(rows_hbm.at[pl.ds(eoff, SC_C)], rows_v)
            pltpu.sync_copy(rows_v, shared.at[lidx_v], add=True)
            return carry

        lax.fori_loop(0, iters, body, 0)
        plsc.subcore_barrier()
        pltpu.sync_copy(shared.at[pl.ds(t * TPT, TPT)],
                        out_hbm.at[c, pl.ds(t * TPT, TPT)])

    return sk


def kernel(x, embeddings, nodes, edges, params, proj, graph):
    n = embeddings.shape[0]
    e = edges.shape[0]
    g0 = graph[0].astype(jnp.int32)
    g1 = graph[1].astype(jnp.int32)

    clusters, means, msq = _run_ka(embeddings, proj)

    gather = _make_gather(e, LATENT)
    src = gather(nodes, g0)
    dst = gather(nodes, g1)

    pe = params["cell_edge"]
    enew = _run_kf(src, dst, edges,
                   pe[0][0], pe[0][1].reshape(1, LATENT),
                   pe[1][0], pe[1][1].reshape(1, LATENT))

    scatter = _make_scatter(e, LATENT)
    agg2 = scatter(enew, g1, jnp.zeros((TPT, LATENT), F32))
    agg = agg2[:, :NH, :].reshape(n, LATENT)

    wb = jnp.abs(params["w_bip"]).reshape(1, 1)
    bidx, bw, sun, den = _run_kb(embeddings, nodes, means,
                                 msq.reshape(1, KC), wb)

    ps = params["supernode_enc"]
    sup = _run_kc(sun, den.reshape(KC, 1), means,
                  ps[0][0], ps[0][1].reshape(1, 64),
                  ps[1][0], ps[1][1].reshape(1, LATENT - EMB))

    pn = params["cell_node"]
    po = params["output"]
    out = _run_kd(nodes, agg, bidx, bw, sup,
                  pn[0][0], pn[0][1].reshape(1, LATENT),
                  pn[1][0], pn[1][1].reshape(1, LATENT),
                  po[0][0], po[0][1].reshape(1, 64),
                  po[1][0], po[1][1].reshape(1, EMB))

    return (out, clusters)


# speedup vs baseline: 6.5740x; 1.0614x over previous
import functools

import jax
import jax.numpy as jnp
from jax import lax
from jax.experimental import pallas as pl
from jax.experimental.pallas import tpu as pltpu
from jax.experimental.pallas import tpu_sc as plsc

LATENT = 64
EMB = 16
KC = 1024
BG_K = 4
EPS = 1e-8

NB = 1000
EB = 8000
NW = 32
GC = 1000
SC_C = 400
NH = 25000
TPT = 1576
NPADT = 16 * TPT
DUMP = 25152

F32 = jnp.float32
BF16 = jnp.bfloat16


def _bdot(a, b, dims=None):
    if dims is None:
        dims = (((a.ndim - 1,), (0,)), ((), ()))
    return lax.dot_general(a.astype(BF16), b.astype(BF16), dims,
                           preferred_element_type=F32)


def _hdot(a, b, dims=None):
    if dims is None:
        dims = (((a.ndim - 1,), (0,)), ((), ()))
    return lax.dot_general(a, b, dims, precision=lax.Precision.HIGHEST,
                           preferred_element_type=F32)


def _ln(h):
    m = jnp.mean(h, axis=-1, keepdims=True)
    v = jnp.mean((h - m) ** 2, axis=-1, keepdims=True)
    return (h - m) / jnp.sqrt(v + 1e-5)


def _relu_ln(h):
    return jnp.maximum(_ln(h), 0.0)


def _ka_body(emb_ref, proj_ref, cl_ref, means_ref, msq_ref, seg_acc, cnt_acc):
    i = pl.program_id(0)

    @pl.when(i == 0)
    def _():
        seg_acc[...] = jnp.zeros_like(seg_acc)
        cnt_acc[...] = jnp.zeros_like(cnt_acc)

    emb = emb_ref[...]
    s = _bdot(emb, proj_ref[...])
    v = jnp.max(s, axis=1, keepdims=True)
    iota = lax.broadcasted_iota(jnp.int32, (NB, KC), 1)
    idx = jnp.min(jnp.where(s >= v, iota, jnp.int32(2 ** 30)), axis=1)
    cl_ref[0, 0, :] = idx
    hot = (iota == idx[:, None]).astype(F32)
    seg_acc[...] += _hdot(hot, emb, (((0,), (0,)), ((), ())))
    cnt_acc[...] += _hdot(hot, jnp.ones((NB, 1), F32), (((0,), (0,)), ((), ())))

    @pl.when(i == pl.num_programs(0) - 1)
    def _():
        cnt = jnp.maximum(cnt_acc[...], 1.0)
        m = seg_acc[...] / cnt
        nrm = jnp.sqrt(jnp.sum(m * m, axis=1, keepdims=True))
        mn = m / (nrm + 1e-12)
        means_ref[...] = mn
        msq_ref[...] = jnp.sum(mn * mn, axis=1, keepdims=True)


def _run_ka(embeddings, proj):
    n = embeddings.shape[0]
    grid = n // NB
    cl3, means, msq = pl.pallas_call(
        _ka_body,
        grid=(grid,),
        in_specs=[
            pl.BlockSpec((NB, EMB), lambda i: (i, 0)),
            pl.BlockSpec((EMB, KC), lambda i: (0, 0)),
        ],
        out_specs=[
            pl.BlockSpec((1, 1, NB), lambda i: (i, 0, 0)),
            pl.BlockSpec((KC, EMB), lambda i: (0, 0)),
            pl.BlockSpec((KC, 1), lambda i: (0, 0)),
        ],
        out_shape=[
            jax.ShapeDtypeStruct((grid, 1, NB), jnp.int32),
            jax.ShapeDtypeStruct((KC, EMB), F32),
            jax.ShapeDtypeStruct((KC, 1), F32),
        ],
        scratch_shapes=[
            pltpu.VMEM((KC, EMB), F32),
            pltpu.VMEM((KC, 1), F32),
        ],
    )(embeddings, proj)
    return cl3.reshape(n), means, msq


def _kb_body(emb_ref, nodes_ref, means_ref, msqr_ref, wb_ref,
             bidx_ref, bw_ref, sun_ref, den_ref):
    i = pl.program_id(0)

    @pl.when(i == 0)
    def _():
        sun_ref[...] = jnp.zeros_like(sun_ref)
        den_ref[...] = jnp.zeros_like(den_ref)

    emb = emb_ref[...]
    means = means_ref[...]
    st = _hdot(emb, means, (((1,), (1,)), ((), ())))
    s = _bdot(emb, means, (((1,), (1,)), ((), ())))
    esq = jnp.sum(emb * emb, axis=1, keepdims=True)
    iota = lax.broadcasted_iota(jnp.int32, (NB, KC), 1)
    wb = wb_ref[0, 0]
    tmat = msqr_ref[...] - 2.0 * st

    sm = s
    facc = jnp.zeros((NB, KC), F32)
    idx_cols = []
    bw_cols = []
    for _ in range(BG_K):
        v = jnp.max(sm, axis=1, keepdims=True)
        idxk = jnp.min(jnp.where(sm >= v, iota, jnp.int32(2 ** 30)),
                       axis=1, keepdims=True)
        hot = (iota == idxk).astype(F32)
        q = jnp.sum(hot * tmat, axis=1, keepdims=True)
        d2 = esq + q
        bwk = jnp.exp(-wb * d2)
        facc = facc + bwk * hot
        idx_cols.append(idxk)
        bw_cols.append(bwk)
        sm = jnp.where(iota == idxk, -jnp.inf, sm)

    bidx_ref[...] = jnp.concatenate(idx_cols, axis=1)
    bw_ref[...] = jnp.concatenate(bw_cols, axis=1)
    sun_ref[...] += _bdot(facc, nodes_ref[...], (((0,), (0,)), ((), ())))
    den_ref[...] += jnp.sum(facc, axis=0, keepdims=True)


def _run_kb(embeddings, nodes, means, msq, wb):
    n = embeddings.shape[0]
    grid = n // NB
    return pl.pallas_call(
        _kb_body,
        grid=(grid,),
        in_specs=[
            pl.BlockSpec((NB, EMB), lambda i: (i, 0)),
            pl.BlockSpec((NB, LATENT), lambda i: (i, 0)),
            pl.BlockSpec((KC, EMB), lambda i: (0, 0)),
            pl.BlockSpec((1, KC), lambda i: (0, 0)),
            pl.BlockSpec((1, 1), lambda i: (0, 0)),
        ],
        out_specs=[
            pl.BlockSpec((NB, BG_K), lambda i: (i, 0)),
            pl.BlockSpec((NB, BG_K), lambda i: (i, 0)),
            pl.BlockSpec((KC, LATENT), lambda i: (0, 0)),
            pl.BlockSpec((1, KC), lambda i: (0, 0)),
        ],
        out_shape=[
            jax.ShapeDtypeStruct((n, BG_K), jnp.int32),
            jax.ShapeDtypeStruct((n, BG_K), F32),
            jax.ShapeDtypeStruct((KC, LATENT), F32),
            jax.ShapeDtypeStruct((1, KC), F32),
        ],
        compiler_params=pltpu.CompilerParams(fuse_transposed_lhs_in_matmul=True),
    )(embeddings, nodes, means, msq, wb)


def _kc_body(sun_ref, den_ref, means_ref, w1_ref, b1_ref, w2_ref, b2_ref,
             sup_ref):
    inv = 1.0 / (den_ref[...] + EPS)
    sn_raw = sun_ref[...] * inv
    h1 = _relu_ln(_bdot(sn_raw, w1_ref[...]) + b1_ref[...])
    h2 = _relu_ln(_bdot(h1, w2_ref[...]) + b2_ref[...])
    sup = jnp.concatenate([means_ref[...], h2], axis=1)
    sup_ref[...] = sup * inv


def _run_kc(sun, den, means, w1, b1, w2, b2):
    return pl.pallas_call(
        _kc_body,
        out_shape=jax.ShapeDtypeStruct((KC, LATENT), F32),
    )(sun, den, means, w1, b1, w2, b2)


def _kf_body(src_ref, dst_ref, edg_ref, w1a_ref, w1b_ref, w1c_ref, b1_ref,
             w2_ref, b2_ref, out_ref):
    edg = edg_ref[...]
    p = (_bdot(src_ref[...], w1a_ref[...]) + _bdot(dst_ref[...], w1b_ref[...])
         + _bdot(edg, w1c_ref[...]) + b1_ref[...])
    h1 = _relu_ln(p)
    h2 = _relu_ln(_bdot(h1, w2_ref[...]) + b2_ref[...])
    out_ref[...] = h2 + edg


def _run_kf(src, dst, edges, w1, b1, w2, b2):
    e = edges.shape[0]
    grid = e // EB
    return pl.pallas_call(
        _kf_body,
        grid=(grid,),
        in_specs=[
            pl.BlockSpec((EB, LATENT), lambda i: (i, 0)),
            pl.BlockSpec((EB, LATENT), lambda i: (i, 0)),
            pl.BlockSpec((EB, LATENT), lambda i: (i, 0)),
            pl.BlockSpec((LATENT, LATENT), lambda i: (0, 0)),
            pl.BlockSpec((LATENT, LATENT), lambda i: (0, 0)),
            pl.BlockSpec((LATENT, LATENT), lambda i: (0, 0)),
            pl.BlockSpec((1, LATENT), lambda i: (0, 0)),
            pl.BlockSpec((LATENT, LATENT), lambda i: (0, 0)),
            pl.BlockSpec((1, LATENT), lambda i: (0, 0)),
        ],
        out_specs=pl.BlockSpec((EB, LATENT), lambda i: (i, 0)),
        out_shape=jax.ShapeDtypeStruct((e, LATENT), F32),
    )(src, dst, edges, w1[:LATENT], w1[LATENT:2 * LATENT], w1[2 * LATENT:],
      b1, w2, b2)


def _kd_body(nodes_ref, agg_ref, bidx_ref, bw_ref, sup_ref,
             w1_ref, b1_ref, w2_ref, b2_ref,
             ow1_ref, ob1_ref, ow2_ref, ob2_ref, out_ref):
    iota = lax.broadcasted_iota(jnp.int32, (NB, KC), 1)
    bidx = bidx_ref[...]
    bw = bw_ref[...]
    facc = jnp.zeros((NB, KC), F32)
    for k in range(BG_K):
        hot = (iota == bidx[:, k:k + 1]).astype(F32)
        facc = facc + bw[:, k:k + 1] * hot
    from_super = _bdot(facc, sup_ref[...])
    nodes = nodes_ref[...]
    h = jnp.concatenate([nodes, agg_ref[...], from_super], axis=1)
    h1 = _relu_ln(_bdot(h, w1_ref[...]) + b1_ref[...])
    h2 = _relu_ln(_bdot(h1, w2_ref[...]) + b2_ref[...])
    nn = h2 + nodes
    o1 = _relu_ln(_bdot(nn, ow1_ref[...]) + ob1_ref[...])
    o2 = _bdot(o1, ow2_ref[...]) + ob2_ref[...]
    nrm = jnp.sqrt(jnp.sum(o2 * o2, axis=1, keepdims=True))
    out_ref[...] = o2 / (nrm + 1e-12)


def _run_kd(nodes, agg, bidx, bw, sup, w1, b1, w2, b2, ow1, ob1, ow2, ob2):
    n = nodes.shape[0]
    grid = n // NB
    return pl.pallas_call(
        _kd_body,
        grid=(grid,),
        in_specs=[
            pl.BlockSpec((NB, LATENT), lambda i: (i, 0)),
            pl.BlockSpec((NB, LATENT), lambda i: (i, 0)),
            pl.BlockSpec((NB, BG_K), lambda i: (i, 0)),
            pl.BlockSpec((NB, BG_K), lambda i: (i, 0)),
            pl.BlockSpec((KC, LATENT), lambda i: (0, 0)),
            pl.BlockSpec((3 * LATENT, LATENT), lambda i: (0, 0)),
            pl.BlockSpec((1, LATENT), lambda i: (0, 0)),
            pl.BlockSpec((LATENT, LATENT), lambda i: (0, 0)),
            pl.BlockSpec((1, LATENT), lambda i: (0, 0)),
            pl.BlockSpec((LATENT, LATENT), lambda i: (0, 0)),
            pl.BlockSpec((1, LATENT), lambda i: (0, 0)),
            pl.BlockSpec((LATENT, EMB), lambda i: (0, 0)),
            pl.BlockSpec((1, EMB), lambda i: (0, 0)),
        ],
        out_specs=pl.BlockSpec((NB, EMB), lambda i: (i, 0)),
        out_shape=jax.ShapeDtypeStruct((n, EMB), F32),
    )(nodes, agg, bidx, bw, sup, w1, b1, w2, b2, ow1, ob1, ow2, ob2)


def _make_gather(e_total, d):
    per_w = e_total // NW
    iters = per_w // GC
    mesh = plsc.VectorSubcoreMesh(core_axis_name="c", subcore_axis_name="s")

    @functools.partial(
        pl.kernel, mesh=mesh,
        out_type=jax.ShapeDtypeStruct((e_total, d), F32),
        scratch_types=[
            pltpu.VMEM((GC,), jnp.int32),
            pltpu.VMEM((GC, d), F32),
            pltpu.SemaphoreType.DMA,
        ],
        compiler_params=pltpu.CompilerParams(use_tc_tiling_on_sc=False),
    )
    def gk(table_hbm, idx_hbm, out_hbm, idx_v, rows_v, sem):
        wid = lax.axis_index("s") * 2 + lax.axis_index("c")

        def body(j, carry):
            base = wid * per_w + j * GC
            pltpu.sync_copy(idx_hbm.at[pl.ds(base, GC)], idx_v)
            pltpu.async_copy(table_hbm.at[idx_v], rows_v, sem).wait()
            pltpu.sync_copy(rows_v, out_hbm.at[pl.ds(base, GC)])
            return carry

        lax.fori_loop(0, iters, body, 0)

    return gk


def _make_scatter(e_total, d):
    per_t = e_total // 16
    iters = per_t // SC_C
    mesh = plsc.VectorSubcoreMesh(core_axis_name="c", subcore_axis_name="s")

    @functools.partial(
        pl.kernel, mesh=mesh,
        out_type=jax.ShapeDtypeStruct((2, NPADT, d), F32),
        scratch_types=[
            pltpu.VMEM((SC_C,), jnp.int32),
            pltpu.VMEM((SC_C,), jnp.int32),
            pltpu.VMEM((SC_C, d), F32),
            pltpu.VMEM_SHARED((NPADT, d), F32),
        ],
        compiler_params=pltpu.CompilerParams(use_tc_tiling_on_sc=False),
    )
    def sk(rows_hbm, g1_hbm, zeros_hbm, out_hbm, gidx_v, lidx_v, rows_v, shared):
        c = lax.axis_index("c")
        t = lax.axis_index("s")
        pltpu.sync_copy(zeros_hbm, shared.at[pl.ds(t * TPT, TPT)])
        plsc.subcore_barrier()
        lo = c * NH

        def body(j, carry):
            eoff = t * per_t + j * SC_C
            pltpu.sync_copy(g1_hbm.at[pl.ds(eoff, SC_C)], gidx_v)

            def ixb(i, cc):
                v = gidx_v[pl.ds(i * 16, 16)]
                l = v - lo
                m = (l >= 0) & (l < NH)
                lidx_v[pl.ds(i * 16, 16)] = jnp.where(m, l, jnp.int32(DUMP))
                return cc

            lax.fori_loop(0, SC_C // 16, ixb, 0)
            pltpu.sync_copy(rows_hbm.at[pl.ds(eoff, SC_C)], rows_v)
            pltpu.sync_copy(rows_v, shared.at[lidx_v], add=True)
            return carry

        lax.fori_loop(0, iters, body, 0)
        plsc.subcore_barrier()
        pltpu.sync_copy(shared.at[pl.ds(t * TPT, TPT)],
                        out_hbm.at[c, pl.ds(t * TPT, TPT)])

    return sk


def kernel(x, embeddings, nodes, edges, params, proj, graph):
    n = embeddings.shape[0]
    e = edges.shape[0]
    g0 = graph[0].astype(jnp.int32)
    g1 = graph[1].astype(jnp.int32)

    clusters, means, msq = _run_ka(embeddings, proj)

    gather = _make_gather(e, LATENT)
    src = gather(nodes, g0)
    dst = gather(nodes, g1)

    pe = params["cell_edge"]
    enew = _run_kf(src, dst, edges,
                   pe[0][0], pe[0][1].reshape(1, LATENT),
                   pe[1][0], pe[1][1].reshape(1, LATENT))

    scatter = _make_scatter(e, LATENT)
    agg2 = scatter(enew, g1, jnp.zeros((TPT, LATENT), F32))
    agg = agg2[:, :NH, :].reshape(n, LATENT)

    wb = jnp.abs(params["w_bip"]).reshape(1, 1)
    bidx, bw, sun, den = _run_kb(embeddings, nodes, means,
                                 msq.reshape(1, KC), wb)

    ps = params["supernode_enc"]
    sup = _run_kc(sun, den.reshape(KC, 1), means,
                  ps[0][0], ps[0][1].reshape(1, 64),
                  ps[1][0], ps[1][1].reshape(1, LATENT - EMB))

    pn = params["cell_node"]
    po = params["output"]
    out = _run_kd(nodes, agg, bidx, bw, sup,
                  pn[0][0], pn[0][1].reshape(1, LATENT),
                  pn[1][0], pn[1][1].reshape(1, LATENT),
                  po[0][0], po[0][1].reshape(1, 64),
                  po[1][0], po[1][1].reshape(1, EMB))

    return (out, clusters)
